# baseline (speedup 1.0000x reference)
import functools

import jax
import jax.numpy as jnp
from jax import lax
from jax.experimental import pallas as pl
from jax.experimental.pallas import tpu as pltpu
from jax.experimental.pallas import tpu_sc as plsc

_NC = 2
_NS = 16
_NW = _NC * _NS


@functools.lru_cache(maxsize=None)
def _make_sc_gather(n_out, n_sc, v, d, c):
    n_per_w = n_sc // _NW
    n_chunks = n_per_w // c
    assert n_chunks >= 4 and n_chunks % 2 == 0
    mesh = plsc.VectorSubcoreMesh(core_axis_name="c", subcore_axis_name="s")

    @functools.partial(
        pl.kernel,
        mesh=mesh,
        out_type=jax.ShapeDtypeStruct((n_out, d), jnp.float32),
        scratch_types=[
            pltpu.VMEM((n_chunks, c), jnp.int32),
            pltpu.VMEM((2, c, d), jnp.float32),
            pltpu.SemaphoreType.DMA,
            pltpu.SemaphoreType.DMA,
            pltpu.SemaphoreType.DMA,
            pltpu.SemaphoreType.DMA,
        ],
    )
    def k(idx_hbm, table_hbm, out_hbm, idx_v, rows_v, g0, g1, w0, w1):
        wid = lax.axis_index("s") * _NC + lax.axis_index("c")
        base = wid * n_per_w
        gsems = (g0, g1)
        wsems = (w0, w1)
        pltpu.sync_copy(idx_hbm.at[wid], idx_v)

        def gath(j, b):
            return pltpu.make_async_copy(
                table_hbm.at[idx_v.at[j]], rows_v.at[b], gsems[b])

        def wr(j, b):
            return pltpu.make_async_copy(
                rows_v.at[b], out_hbm.at[pl.ds(base + j * c, c)], wsems[b])

        gath(0, 0).start()
        gath(0, 0).wait()
        wr(0, 0).start()
        gath(1, 1).start()

        def step(j, b):
            gath(j, b).wait()
            wr(j, b).start()
            wr(j - 1, 1 - b).wait()
            gath(j + 1, 1 - b).start()

        def body(i, carry):
            step(2 * i + 1, 1)
            step(2 * i + 2, 0)
            return carry

        lax.fori_loop(0, (n_chunks - 2) // 2, body, 0)
        j_last = n_chunks - 1
        gath(j_last, 1).wait()
        wr(j_last, 1).start()
        wr(j_last - 1, 0).wait()
        wr(j_last, 1).wait()

    return k


@functools.lru_cache(maxsize=None)
def _make_tc_gather(n, v, d, g):
    def body(idx_ref, pe_ref, o_ref):
        def cp(r, carry):
            row = idx_ref[r]
            o_ref[pl.ds(r, 1), :] = pe_ref[pl.ds(row, 1), :]
            return carry

        lax.fori_loop(0, g, cp, 0)

    return pl.pallas_call(
        body,
        grid=(n // g,),
        in_specs=[
            pl.BlockSpec((g,), lambda i: (i,), memory_space=pltpu.SMEM),
            pl.BlockSpec((v, d), lambda i: (0, 0)),
        ],
        out_specs=pl.BlockSpec((g, d), lambda i: (i, 0)),
        out_shape=jax.ShapeDtypeStruct((n, d), jnp.float32),
    )


def kernel(x, pe):
    b, s = x.shape
    v, d = pe.shape
    n = b * s
    c = 32
    n_sc = (3 * n // 4 // (_NW * c * 2)) * (_NW * c * 2)
    n_tc = n - n_sc
    flat = x.reshape(n).astype(jnp.int32)
    idx_sc = flat[:n_sc].reshape(_NW, (n_sc // _NW) // c, c)
    out_full = _make_sc_gather(n, n_sc, v, d, c)(idx_sc, pe)
    out_tc = _make_tc_gather(n_tc, v, d, 256)(flat[n_sc:], pe)
    out = lax.dynamic_update_slice(out_full, out_tc, (n_sc, 0))
    return out.reshape(b, s, d)


# speedup vs baseline: 1.2106x; 1.2106x over previous
import functools

import jax
import jax.numpy as jnp
from jax import lax
from jax.experimental import pallas as pl
from jax.experimental.pallas import tpu as pltpu
from jax.experimental.pallas import tpu_sc as plsc

_NC = 2
_NS = 16
_NW = _NC * _NS


@functools.lru_cache(maxsize=None)
def _make_gather(n, v, d, c):
    n_per_w = n // _NW
    n_chunks = n_per_w // c
    n_super = n_chunks // 2
    assert n_chunks % 2 == 0 and n_super >= 4 and n_super % 2 == 0
    mesh = plsc.VectorSubcoreMesh(core_axis_name="c", subcore_axis_name="s")

    @functools.partial(
        pl.kernel,
        mesh=mesh,
        out_type=jax.ShapeDtypeStruct((n, d), jnp.float32),
        scratch_types=[
            pltpu.VMEM((n_chunks, c), jnp.int32),
            pltpu.VMEM((4, c, d), jnp.float32),
            pltpu.SemaphoreType.DMA,
            pltpu.SemaphoreType.DMA,
            pltpu.SemaphoreType.DMA,
            pltpu.SemaphoreType.DMA,
        ],
    )
    def k(idx_hbm, table_hbm, out_hbm, idx_v, rows_v, ga, gb, wa, wb):
        wid = lax.axis_index("s") * _NC + lax.axis_index("c")
        base = wid * n_per_w
        gsems = (ga, gb)
        wsems = (wa, wb)
        pltpu.sync_copy(idx_hbm.at[wid], idx_v)

        def gath(j, buf, grp):
            return pltpu.make_async_copy(
                table_hbm.at[idx_v.at[j]], rows_v.at[buf], gsems[grp])

        def wr(j, buf, grp):
            return pltpu.make_async_copy(
                rows_v.at[buf], out_hbm.at[pl.ds(base + j * c, c)],
                wsems[grp])

        def fire_g(i, grp):
            gath(2 * i, 2 * grp, grp).start()
            gath(2 * i + 1, 2 * grp + 1, grp).start()

        def sstep(i, grp, first=False, last=False):
            oth = 1 - grp
            gath(2 * i, 2 * grp, grp).wait()
            gath(2 * i + 1, 2 * grp + 1, grp).wait()
            wr(2 * i, 2 * grp, grp).start()
            wr(2 * i + 1, 2 * grp + 1, grp).start()
            if not first:
                wr(2 * i - 2, 2 * oth, oth).wait()
                wr(2 * i - 1, 2 * oth + 1, oth).wait()
            if not last:
                fire_g(i + 1, oth)

        fire_g(0, 0)
        sstep(0, 0, first=True)

        def body(t, carry):
            i = 2 * t + 1
            sstep(i, 1)
            sstep(i + 1, 0)
            return carry

        lax.fori_loop(0, (n_super - 2) // 2, body, 0)
        i_last = n_super - 1
        sstep(i_last, 1, last=True)
        wr(2 * i_last, 2, 1).wait()
        wr(2 * i_last + 1, 3, 1).wait()

    return k


def kernel(x, pe):
    b, s = x.shape
    v, d = pe.shape
    n = b * s
    c = 16
    idx = x.reshape(_NW, (n // _NW) // c, c).astype(jnp.int32)
    out = _make_gather(n, v, d, c)(idx, pe)
    return out.reshape(b, s, d)
